# baseline (speedup 1.0000x reference)
import functools

import jax
import jax.numpy as jnp
import numpy as np
from jax import lax
from jax.experimental import pallas as pl
from jax.experimental.pallas import tpu as pltpu
from jax.experimental.pallas import tpu_sc as plsc

D_MODEL = 768
MAX_S = 4096

try:
    _info = plsc.get_sparse_core_info()
    _NC, _NS = _info.num_cores, _info.num_subcores
except ValueError:
    _NC, _NS = 2, 16
_NW = _NC * _NS


def _pe_body(o_ref):
    rows = o_ref.shape[0]
    hi_n = rows // 64
    base = pl.program_id(0) * rows

    j_hi = jax.lax.broadcasted_iota(jnp.int32, (hi_n, D_MODEL), 1)
    k_hi = (j_hi // 2).astype(jnp.float32)
    div_hi = jnp.exp(k_hi * (-2.0 * jnp.log(10000.0) / D_MODEL))
    pos_hi = (jax.lax.broadcasted_iota(jnp.int32, (hi_n, D_MODEL), 0) * 64
              + base).astype(jnp.float32)
    ang_a = pos_hi * div_hi
    sin_a, cos_a = jnp.sin(ang_a), jnp.cos(ang_a)

    j_lo = jax.lax.broadcasted_iota(jnp.int32, (64, D_MODEL), 1)
    k_lo = (j_lo // 2).astype(jnp.float32)
    div_lo = jnp.exp(k_lo * (-2.0 * jnp.log(10000.0) / D_MODEL))
    pos_lo = jax.lax.broadcasted_iota(jnp.int32, (64, D_MODEL), 0)
    phase = (j_lo % 2).astype(jnp.float32) * (0.5 * jnp.pi)
    ang_b = pos_lo.astype(jnp.float32) * div_lo + phase
    sin_b, cos_b = jnp.sin(ang_b), jnp.cos(ang_b)

    pe = (sin_a[:, None, :] * cos_b[None, :, :]
          + cos_a[:, None, :] * sin_b[None, :, :])
    o_ref[...] = pe.reshape(rows, D_MODEL)


def _make_pe(seq_len):
    rows = 512
    return pl.pallas_call(
        _pe_body,
        grid=(seq_len // rows,),
        out_specs=pl.BlockSpec((rows, D_MODEL), lambda i: (i, 0)),
        out_shape=jax.ShapeDtypeStruct((seq_len, D_MODEL), jnp.float32),
    )()


@functools.lru_cache(maxsize=2)
def _pe_const(seq_len):
    pos = np.arange(seq_len, dtype=np.float64)[:, None]
    i = np.arange(0, D_MODEL, 2, dtype=np.float64)
    div = np.exp(i * (-np.log(10000.0) / D_MODEL))
    ang = pos * div[None, :]
    pe = np.zeros((seq_len, D_MODEL), dtype=np.float64)
    pe[:, 0::2] = np.sin(ang)
    pe[:, 1::2] = np.cos(ang)
    return pe.astype(np.float32)


def _make_emb(B, V, D, seq_len):
    assert seq_len % _NW == 0
    s_w = seq_len // _NW
    Cs = 8
    NBUF = 3
    assert s_w % Cs == 0
    nchunk = s_w // Cs
    R = B * Cs

    mesh = plsc.VectorSubcoreMesh(core_axis_name="c", subcore_axis_name="s")

    @functools.partial(
        pl.kernel,
        mesh=mesh,
        out_type=jax.ShapeDtypeStruct((B * seq_len, D), jnp.float32),
        scratch_types=(
            [pltpu.VMEM((B, s_w), jnp.int32)]
            + [pltpu.VMEM((R, D), jnp.float32) for _ in range(NBUF)]
            + [pltpu.VMEM((Cs, D), jnp.float32) for _ in range(NBUF)]
            + [pltpu.SemaphoreType.DMA for _ in range(2 * NBUF)]
        ),
    )
    def emb(x_hbm, table_hbm, pe_hbm, out_hbm, idx_v, *rest):
        gbufs = rest[:NBUF]
        pbufs = rest[NBUF:2 * NBUF]
        gsems = rest[2 * NBUF:3 * NBUF]
        ssems = rest[3 * NBUF:4 * NBUF]
        wid = lax.axis_index("s") * _NC + lax.axis_index("c")
        s_base = wid * s_w
        pltpu.sync_copy(x_hbm.at[:, pl.ds(s_base, s_w)], idx_v)

        def issue(c, k):
            ds = []
            for b in range(B):
                ds.append(pltpu.async_copy(
                    table_hbm.at[idx_v.at[b, pl.ds(c * Cs, Cs)]],
                    gbufs[k].at[pl.ds(b * Cs, Cs)], gsems[k]))
            ds.append(pltpu.async_copy(
                pe_hbm.at[pl.ds(s_base + c * Cs, Cs)], pbufs[k], gsems[k]))
            return ds

        store_desc = [None] * NBUF
        descs = [None] * NBUF
        for c in range(min(NBUF - 1, nchunk)):
            descs[c] = issue(c, c)
        for c in range(nchunk):
            k = c % NBUF
            cn = c + NBUF - 1
            if cn < nchunk:
                nk = cn % NBUF
                if store_desc[nk] is not None:
                    for sd in store_desc[nk]:
                        sd.wait()
                    store_desc[nk] = None
                descs[nk] = issue(cn, nk)
            for d in descs[k]:
                d.wait()
            g, p = gbufs[k], pbufs[k]

            @plsc.parallel_loop(0, Cs)
            def _row(r):
                @plsc.parallel_loop(0, D, step=16, unroll=4)
                def _col(j):
                    pv = p[r, pl.ds(j, 16)]
                    for b in range(B):
                        g[b * Cs + r, pl.ds(j, 16)] = (
                            g[b * Cs + r, pl.ds(j, 16)] + pv)

            sds = []
            for b in range(B):
                sds.append(pltpu.async_copy(
                    g.at[pl.ds(b * Cs, Cs)],
                    out_hbm.at[pl.ds(b * seq_len + s_base + c * Cs, Cs)],
                    ssems[k]))
            store_desc[k] = sds
        for k in range(NBUF):
            if store_desc[k] is not None:
                for sd in store_desc[k]:
                    sd.wait()

    return emb


def kernel(x, table):
    B, S = x.shape
    V, D = table.shape
    pe = jnp.asarray(_pe_const(S))
    out = _make_emb(B, V, D, S)(x.astype(jnp.int32), table, pe)
    return out.reshape(B, S, D)


# speedup vs baseline: 1.0026x; 1.0026x over previous
import functools

import jax
import jax.numpy as jnp
from jax import lax
from jax.experimental import pallas as pl
from jax.experimental.pallas import tpu as pltpu
from jax.experimental.pallas import tpu_sc as plsc

D_MODEL = 768
MAX_S = 4096

try:
    _info = plsc.get_sparse_core_info()
    _NC, _NS = _info.num_cores, _info.num_subcores
except ValueError:
    _NC, _NS = 2, 16
_NW = _NC * _NS


def _pe_body(o_ref):
    rows = o_ref.shape[0]
    hi_n = rows // 64
    base = pl.program_id(0) * rows

    j_hi = jax.lax.broadcasted_iota(jnp.int32, (hi_n, D_MODEL), 1)
    k_hi = (j_hi // 2).astype(jnp.float32)
    div_hi = jnp.exp(k_hi * (-2.0 * jnp.log(10000.0) / D_MODEL))
    pos_hi = (jax.lax.broadcasted_iota(jnp.int32, (hi_n, D_MODEL), 0) * 64
              + base).astype(jnp.float32)
    ang_a = pos_hi * div_hi
    sin_a, cos_a = jnp.sin(ang_a), jnp.cos(ang_a)

    j_lo = jax.lax.broadcasted_iota(jnp.int32, (64, D_MODEL), 1)
    k_lo = (j_lo // 2).astype(jnp.float32)
    div_lo = jnp.exp(k_lo * (-2.0 * jnp.log(10000.0) / D_MODEL))
    pos_lo = jax.lax.broadcasted_iota(jnp.int32, (64, D_MODEL), 0)
    phase = (j_lo % 2).astype(jnp.float32) * (0.5 * jnp.pi)
    ang_b = pos_lo.astype(jnp.float32) * div_lo + phase
    sin_b, cos_b = jnp.sin(ang_b), jnp.cos(ang_b)

    pe = (sin_a[:, None, :] * cos_b[None, :, :]
          + cos_a[:, None, :] * sin_b[None, :, :])
    o_ref[...] = pe.reshape(rows, D_MODEL)


def _make_pe(seq_len):
    rows = 512
    return pl.pallas_call(
        _pe_body,
        grid=(seq_len // rows,),
        out_specs=pl.BlockSpec((rows, D_MODEL), lambda i: (i, 0)),
        out_shape=jax.ShapeDtypeStruct((seq_len, D_MODEL), jnp.float32),
    )()


def _make_emb(B, V, D, seq_len):
    assert seq_len % _NW == 0
    s_w = seq_len // _NW
    Cs = 8
    NBUF = 3
    assert s_w % Cs == 0
    nchunk = s_w // Cs
    R = B * Cs

    mesh = plsc.VectorSubcoreMesh(core_axis_name="c", subcore_axis_name="s")

    @functools.partial(
        pl.kernel,
        mesh=mesh,
        out_type=jax.ShapeDtypeStruct((B * seq_len, D), jnp.float32),
        scratch_types=(
            [pltpu.VMEM((B, s_w), jnp.int32)]
            + [pltpu.VMEM((R, D), jnp.float32) for _ in range(NBUF)]
            + [pltpu.VMEM((Cs, D), jnp.float32) for _ in range(NBUF)]
            + [pltpu.SemaphoreType.DMA for _ in range(2 * NBUF)]
        ),
    )
    def emb(x_hbm, table_hbm, pe_hbm, out_hbm, idx_v, *rest):
        gbufs = rest[:NBUF]
        pbufs = rest[NBUF:2 * NBUF]
        gsems = rest[2 * NBUF:3 * NBUF]
        ssems = rest[3 * NBUF:4 * NBUF]
        wid = lax.axis_index("s") * _NC + lax.axis_index("c")
        s_base = wid * s_w
        pltpu.sync_copy(x_hbm.at[:, pl.ds(s_base, s_w)], idx_v)

        def issue(c, k):
            ds = []
            for b in range(B):
                ds.append(pltpu.async_copy(
                    table_hbm.at[idx_v.at[b, pl.ds(c * Cs, Cs)]],
                    gbufs[k].at[pl.ds(b * Cs, Cs)], gsems[k]))
            ds.append(pltpu.async_copy(
                pe_hbm.at[pl.ds(s_base + c * Cs, Cs)], pbufs[k], gsems[k]))
            return ds

        store_desc = [None] * NBUF
        descs = [None] * NBUF
        for c in range(min(NBUF - 1, nchunk)):
            descs[c] = issue(c, c)
        for c in range(nchunk):
            k = c % NBUF
            cn = c + NBUF - 1
            if cn < nchunk:
                nk = cn % NBUF
                if store_desc[nk] is not None:
                    for sd in store_desc[nk]:
                        sd.wait()
                    store_desc[nk] = None
                descs[nk] = issue(cn, nk)
            for d in descs[k]:
                d.wait()
            g, p = gbufs[k], pbufs[k]

            @plsc.parallel_loop(0, Cs)
            def _row(r):
                @plsc.parallel_loop(0, D, step=16, unroll=4)
                def _col(j):
                    pv = p[r, pl.ds(j, 16)]
                    for b in range(B):
                        g[b * Cs + r, pl.ds(j, 16)] = (
                            g[b * Cs + r, pl.ds(j, 16)] + pv)

            sds = []
            for b in range(B):
                sds.append(pltpu.async_copy(
                    g.at[pl.ds(b * Cs, Cs)],
                    out_hbm.at[pl.ds(b * seq_len + s_base + c * Cs, Cs)],
                    ssems[k]))
            store_desc[k] = sds
        for k in range(NBUF):
            if store_desc[k] is not None:
                for sd in store_desc[k]:
                    sd.wait()

    return emb


def kernel(x, table):
    B, S = x.shape
    V, D = table.shape
    pe = _make_pe(S)
    out = _make_emb(B, V, D, S)(x.astype(jnp.int32), table, pe)
    return out.reshape(B, S, D)
